# Initial kernel scaffold; baseline (speedup 1.0000x reference)
import jax
import jax.numpy as jnp
from jax.experimental import pallas as pl


def kernel(x_nchw, mean_l, t1, b1, t2, b2, t3, b3):
    raise NotImplementedError("write your pallas kernel here")


# speedup vs baseline: 1.0682x; 1.0682x over previous
import jax
import jax.numpy as jnp
from jax.experimental import pallas as pl
from jax.experimental.pallas import tpu as pltpu


H = W = 16
C0, C1, C2 = 3, 8, 16
HP, WP = H // 2, W // 2
_NB = 256


def _conv_block(a, w_ref, b_ref, relu):
    nb, rows, kin = a.shape
    kout = w_ref.shape[1] // 3
    z = jnp.dot(a.reshape(nb * rows, kin), w_ref[...],
                preferred_element_type=jnp.float32)
    z = z.reshape(nb, rows, 3 * kout)
    z0 = z[..., :kout]
    z1 = z[..., kout:2 * kout]
    z2 = z[..., 2 * kout:]
    zero = jnp.zeros((nb, 1, kout), jnp.float32)
    acc = (z1
           + jnp.concatenate([zero, z0[:, :rows - 1]], axis=1)
           + jnp.concatenate([z2[:, 1:], zero], axis=1)
           + b_ref[...])
    if relu:
        acc = jnp.maximum(acc, 0.0)
    return acc


def _fwd_kernel(x_ref, mean_ref, w1_ref, b1_ref, w2_ref, b2_ref,
                w3_ref, b3_ref, o_ref):
    nb = x_ref.shape[0]
    x = x_ref[...]

    lhs = jnp.concatenate([x[:, c] for c in range(C0)], axis=-1)
    lhs = (lhs - mean_ref[...]).astype(jnp.bfloat16)

    a1 = _conv_block(lhs, w1_ref, b1_ref, relu=True).astype(jnp.bfloat16)
    a2 = _conv_block(a1, w2_ref, b2_ref, relu=True).astype(jnp.bfloat16)

    r = a2.reshape(nb, HP, 2, C1 * W)
    mh = jnp.maximum(r[:, :, 0, :], r[:, :, 1, :])
    mh_shift = jnp.concatenate([mh[..., 1:], mh[..., :1]], axis=-1)
    mw = jnp.maximum(mh, mh_shift)

    a3 = _conv_block(mw, w3_ref, b3_ref, relu=False)
    o_ref[...] = a3.astype(o_ref.dtype)


def kernel(x_nchw, mean_l, t1, b1, t2, b2, t3, b3):
    N = x_nchw.shape[0]
    nb = min(_NB, N)
    nblocks = pl.cdiv(N, nb)
    npad = nblocks * nb
    if npad != N:
        x_nchw = jnp.pad(x_nchw, ((0, npad - N), (0, 0), (0, 0), (0, 0)))

    w1 = jnp.concatenate([t1[0], t1[1], t1[2]], axis=1).astype(jnp.bfloat16)
    w2 = jnp.concatenate([t2[0], t2[1], t2[2]], axis=1).astype(jnp.bfloat16)
    w3 = jnp.concatenate([t3[0], t3[1], t3[2]], axis=1).astype(jnp.bfloat16)

    out = pl.pallas_call(
        _fwd_kernel,
        out_shape=jax.ShapeDtypeStruct((npad, HP, C2 * WP), x_nchw.dtype),
        grid=(nblocks,),
        in_specs=[
            pl.BlockSpec((nb, C0, H, W), lambda n: (n, 0, 0, 0)),
            pl.BlockSpec((1, 1, C0 * W), lambda n: (0, 0, 0)),
            pl.BlockSpec((C0 * W, 3 * C1 * W), lambda n: (0, 0)),
            pl.BlockSpec((1, C1 * W), lambda n: (0, 0)),
            pl.BlockSpec((C1 * W, 3 * C1 * W), lambda n: (0, 0)),
            pl.BlockSpec((1, C1 * W), lambda n: (0, 0)),
            pl.BlockSpec((C1 * W, 3 * C2 * WP), lambda n: (0, 0)),
            pl.BlockSpec((1, C2 * WP), lambda n: (0, 0)),
        ],
        out_specs=pl.BlockSpec((nb, HP, C2 * WP), lambda n: (n, 0, 0)),
        compiler_params=pltpu.CompilerParams(
            dimension_semantics=("parallel",)),
    )(x_nchw, mean_l, w1, b1, w2, b2, w3, b3)

    out = out[:N]
    return jnp.transpose(out.reshape(N, HP, C2, WP), (0, 2, 1, 3))
